# baseline (device time: 567812 ns/iter reference)
import jax
import jax.numpy as jnp
from jax import lax
from jax.experimental import pallas as pl
from jax.experimental.pallas import tpu as pltpu

M = 4096
MH = M // 2
N = 4096
K = 8192
NTILES = 16
NT = N // NTILES
KT = 512
KSTEPS = K // KT


def kernel(dy, W):
    my_y = lax.axis_index("y")

    def body(yb_ref, dy_ref, w_ref, out_ref, acc_ref, rbuf_ref,
             p1_send, p1_recv, p2_send, p2_recv, loc_sem):
        n = pl.program_id(0)
        k = pl.program_id(1)
        slot = lax.rem(n, 2)

        prod = lax.dot_general(
            dy_ref[...], w_ref[...],
            dimension_numbers=(((1,), (1,)), ((), ())),
            preferred_element_type=jnp.float32,
        )

        @pl.when(k == 0)
        def _init():
            acc_ref[slot] = prod

        @pl.when(k > 0)
        def _accum():
            acc_ref[slot] += prod

        @pl.when(k == KSTEPS - 1)
        def _comm():
            mx = lax.axis_index("x")
            my = lax.axis_index("y")
            x_peer = (1 - mx, my)
            y_peer = (mx, 1 - my)
            row0 = my * MH

            def p1_desc(t, tslot):
                return pltpu.make_async_remote_copy(
                    src_ref=acc_ref.at[tslot], dst_ref=rbuf_ref.at[t],
                    send_sem=p1_send.at[t], recv_sem=p1_recv.at[t],
                    device_id=x_peer, device_id_type=pl.DeviceIdType.MESH)

            def p2_desc(t):
                return pltpu.make_async_remote_copy(
                    src_ref=rbuf_ref.at[t],
                    dst_ref=out_ref.at[pl.ds(row0, MH), pl.ds(t * NT, NT)],
                    send_sem=p2_send.at[t], recv_sem=p2_recv.at[t],
                    device_id=y_peer, device_id_type=pl.DeviceIdType.MESH)

            def loc_desc(t):
                return pltpu.make_async_copy(
                    rbuf_ref.at[t],
                    out_ref.at[pl.ds(row0, MH), pl.ds(t * NT, NT)],
                    loc_sem.at[t])

            def finish_tile(t, tslot):
                p1_desc(t, tslot).wait()
                rbuf_ref[t] = rbuf_ref[t] + acc_ref[tslot]
                loc_desc(t).start()
                p2_desc(t).start()

            @pl.when(n == 0)
            def _barrier():
                barrier = pltpu.get_barrier_semaphore()
                for nbr in (x_peer, y_peer):
                    pl.semaphore_signal(barrier, inc=1, device_id=nbr,
                                        device_id_type=pl.DeviceIdType.MESH)
                pl.semaphore_wait(barrier, 2)

            p1_desc(n, slot).start()

            @pl.when(n > 0)
            def _finish_prev():
                finish_tile(n - 1, lax.rem(n - 1, 2))

            @pl.when(n == NTILES - 1)
            def _epilogue():
                finish_tile(n, slot)
                for t in range(NTILES):
                    loc_desc(t).wait()
                    p2_desc(t).wait_send()
                    p2_desc(t).wait_recv()

        return None

    grid_spec = pltpu.PrefetchScalarGridSpec(
        num_scalar_prefetch=1,
        grid=(NTILES, KSTEPS),
        in_specs=[
            pl.BlockSpec((MH, KT), lambda n, k, yb: (yb[0], k)),
            pl.BlockSpec((NT, KT), lambda n, k, yb: (n, k)),
        ],
        out_specs=pl.BlockSpec(memory_space=pl.ANY),
        scratch_shapes=[
            pltpu.VMEM((2, MH, NT), jnp.float32),
            pltpu.VMEM((NTILES, MH, NT), jnp.float32),
            pltpu.SemaphoreType.DMA((NTILES,)),
            pltpu.SemaphoreType.DMA((NTILES,)),
            pltpu.SemaphoreType.DMA((NTILES,)),
            pltpu.SemaphoreType.DMA((NTILES,)),
            pltpu.SemaphoreType.DMA((NTILES,)),
        ],
    )
    return pl.pallas_call(
        body,
        grid_spec=grid_spec,
        out_shape=jax.ShapeDtypeStruct((M, N), jnp.float32),
        compiler_params=pltpu.CompilerParams(
            collective_id=0,
            vmem_limit_bytes=60 * 1024 * 1024,
        ),
    )(my_y[None].astype(jnp.int32), dy, W)


# device time: 485233 ns/iter; 1.1702x vs baseline; 1.1702x over previous
import jax
import jax.numpy as jnp
from jax import lax
from jax.experimental import pallas as pl
from jax.experimental.pallas import tpu as pltpu

M = 4096
MH = M // 2
N = 4096
K = 8192
NTILES = 8
NT = N // NTILES
KT = 512
KSTEPS = K // KT
HH = MH // 2


def kernel(dy, W):
    my_y = lax.axis_index("y")

    def body(yb_ref, dy_ref, w_ref, out_ref, acc_ref, rbuf_ref,
             p1_send, p1_recv, p2_send, p2_recv, loc_sem):
        n = pl.program_id(0)
        k = pl.program_id(1)
        slot = lax.rem(n, 2)

        prod = lax.dot_general(
            dy_ref[...], w_ref[...],
            dimension_numbers=(((1,), (1,)), ((), ())),
            preferred_element_type=jnp.float32,
        )

        @pl.when(k == 0)
        def _init():
            acc_ref[slot] = prod

        @pl.when(k > 0)
        def _accum():
            acc_ref[slot] += prod

        @pl.when(k == KSTEPS - 1)
        def _comm():
            mx = lax.axis_index("x")
            my = lax.axis_index("y")
            x_peer = (1 - mx, my)
            y_peer = (mx, 1 - my)
            row0 = my * MH

            def p1_desc(t, tslot, h):
                return pltpu.make_async_remote_copy(
                    src_ref=acc_ref.at[tslot, pl.ds(h * HH, HH)],
                    dst_ref=rbuf_ref.at[t, pl.ds(h * HH, HH)],
                    send_sem=p1_send.at[t, h], recv_sem=p1_recv.at[t, h],
                    device_id=x_peer, device_id_type=pl.DeviceIdType.MESH)

            def p2_desc(t, h):
                return pltpu.make_async_remote_copy(
                    src_ref=rbuf_ref.at[t, pl.ds(h * HH, HH)],
                    dst_ref=out_ref.at[pl.ds(row0 + h * HH, HH),
                                       pl.ds(t * NT, NT)],
                    send_sem=p2_send.at[t, h], recv_sem=p2_recv.at[t, h],
                    device_id=y_peer, device_id_type=pl.DeviceIdType.MESH)

            def loc_desc(t, h):
                return pltpu.make_async_copy(
                    rbuf_ref.at[t, pl.ds(h * HH, HH)],
                    out_ref.at[pl.ds(row0 + h * HH, HH), pl.ds(t * NT, NT)],
                    loc_sem.at[t, h])

            def finish_tile(t, tslot):
                for h in range(2):
                    p1_desc(t, tslot, h).wait()
                    rbuf_ref[t, pl.ds(h * HH, HH)] = (
                        rbuf_ref[t, pl.ds(h * HH, HH)]
                        + acc_ref[tslot, pl.ds(h * HH, HH)])
                    loc_desc(t, h).start()
                    p2_desc(t, h).start()

            @pl.when(n == 0)
            def _barrier():
                barrier = pltpu.get_barrier_semaphore()
                for nbr in (x_peer, y_peer):
                    pl.semaphore_signal(barrier, inc=1, device_id=nbr,
                                        device_id_type=pl.DeviceIdType.MESH)
                pl.semaphore_wait(barrier, 2)

            for h in range(2):
                p1_desc(n, slot, h).start()

            @pl.when(n > 0)
            def _finish_prev():
                finish_tile(n - 1, lax.rem(n - 1, 2))

            @pl.when(n == NTILES - 1)
            def _epilogue():
                finish_tile(n, slot)
                for t in range(NTILES):
                    for h in range(2):
                        loc_desc(t, h).wait()
                        p2_desc(t, h).wait_send()
                        p2_desc(t, h).wait_recv()

        return None

    grid_spec = pltpu.PrefetchScalarGridSpec(
        num_scalar_prefetch=1,
        grid=(NTILES, KSTEPS),
        in_specs=[
            pl.BlockSpec((MH, KT), lambda n, k, yb: (yb[0], k)),
            pl.BlockSpec((NT, KT), lambda n, k, yb: (n, k)),
        ],
        out_specs=pl.BlockSpec(memory_space=pl.ANY),
        scratch_shapes=[
            pltpu.VMEM((2, MH, NT), jnp.float32),
            pltpu.VMEM((NTILES, MH, NT), jnp.float32),
            pltpu.SemaphoreType.DMA((NTILES, 2)),
            pltpu.SemaphoreType.DMA((NTILES, 2)),
            pltpu.SemaphoreType.DMA((NTILES, 2)),
            pltpu.SemaphoreType.DMA((NTILES, 2)),
            pltpu.SemaphoreType.DMA((NTILES, 2)),
        ],
    )
    return pl.pallas_call(
        body,
        grid_spec=grid_spec,
        out_shape=jax.ShapeDtypeStruct((M, N), jnp.float32),
        compiler_params=pltpu.CompilerParams(
            collective_id=0,
            vmem_limit_bytes=60 * 1024 * 1024,
        ),
    )(my_y[None].astype(jnp.int32), dy, W)


# device time: 473663 ns/iter; 1.1988x vs baseline; 1.0244x over previous
import jax
import jax.numpy as jnp
from jax import lax
from jax.experimental import pallas as pl
from jax.experimental.pallas import tpu as pltpu

M = 4096
MH = M // 2
N = 4096
K = 8192
NTILES = 8
NT = N // NTILES
KT = 1024
KSTEPS = K // KT
HH = MH // 2
RSLOTS = 6


def kernel(dy, W):
    my_y = lax.axis_index("y")

    def body(yb_ref, dy_ref, w_ref, out_ref, acc_ref, rbuf_ref,
             p1_send, p1_recv, p2_send, p2_recv, loc_sem):
        n = pl.program_id(0)
        k = pl.program_id(1)
        slot = lax.rem(n, 2)

        prod = lax.dot_general(
            dy_ref[...], w_ref[...],
            dimension_numbers=(((1,), (1,)), ((), ())),
            preferred_element_type=jnp.float32,
        )

        @pl.when(k == 0)
        def _init():
            acc_ref[slot] = prod

        @pl.when(k > 0)
        def _accum():
            acc_ref[slot] += prod

        @pl.when(k == KSTEPS - 1)
        def _comm():
            mx = lax.axis_index("x")
            my = lax.axis_index("y")
            x_peer = (1 - mx, my)
            y_peer = (mx, 1 - my)
            row0 = my * MH

            def p1_desc(t, tslot, h):
                return pltpu.make_async_remote_copy(
                    src_ref=acc_ref.at[tslot, pl.ds(h * HH, HH)],
                    dst_ref=rbuf_ref.at[lax.rem(t, RSLOTS), pl.ds(h * HH, HH)],
                    send_sem=p1_send.at[t, h], recv_sem=p1_recv.at[t, h],
                    device_id=x_peer, device_id_type=pl.DeviceIdType.MESH)

            def p2_desc(t, h):
                return pltpu.make_async_remote_copy(
                    src_ref=rbuf_ref.at[lax.rem(t, RSLOTS), pl.ds(h * HH, HH)],
                    dst_ref=out_ref.at[pl.ds(row0 + h * HH, HH),
                                       pl.ds(t * NT, NT)],
                    send_sem=p2_send.at[t, h], recv_sem=p2_recv.at[t, h],
                    device_id=y_peer, device_id_type=pl.DeviceIdType.MESH)

            def loc_desc(t, h):
                return pltpu.make_async_copy(
                    rbuf_ref.at[lax.rem(t, RSLOTS), pl.ds(h * HH, HH)],
                    out_ref.at[pl.ds(row0 + h * HH, HH), pl.ds(t * NT, NT)],
                    loc_sem.at[t, h])

            def finish_tile(t, tslot):
                rs = lax.rem(t, RSLOTS)
                for h in range(2):
                    p1_desc(t, tslot, h).wait()
                    rbuf_ref[rs, pl.ds(h * HH, HH)] = (
                        rbuf_ref[rs, pl.ds(h * HH, HH)]
                        + acc_ref[tslot, pl.ds(h * HH, HH)])
                    loc_desc(t, h).start()
                    p2_desc(t, h).start()

            @pl.when(n == 0)
            def _barrier():
                barrier = pltpu.get_barrier_semaphore()
                for nbr in (x_peer, y_peer):
                    pl.semaphore_signal(barrier, inc=1, device_id=nbr,
                                        device_id_type=pl.DeviceIdType.MESH)
                pl.semaphore_wait(barrier, 2)

            for h in range(2):
                p1_desc(n, slot, h).start()

            @pl.when(n > 0)
            def _finish_prev():
                finish_tile(n - 1, lax.rem(n - 1, 2))

            @pl.when(n == NTILES - 1)
            def _epilogue():
                finish_tile(n, slot)
                for t in range(NTILES):
                    for h in range(2):
                        loc_desc(t, h).wait()
                        p2_desc(t, h).wait_send()
                        p2_desc(t, h).wait_recv()

        return None

    grid_spec = pltpu.PrefetchScalarGridSpec(
        num_scalar_prefetch=1,
        grid=(NTILES, KSTEPS),
        in_specs=[
            pl.BlockSpec((MH, KT), lambda n, k, yb: (yb[0], k)),
            pl.BlockSpec((NT, KT), lambda n, k, yb: (n, k)),
        ],
        out_specs=pl.BlockSpec(memory_space=pl.ANY),
        scratch_shapes=[
            pltpu.VMEM((2, MH, NT), jnp.float32),
            pltpu.VMEM((RSLOTS, MH, NT), jnp.float32),
            pltpu.SemaphoreType.DMA((NTILES, 2)),
            pltpu.SemaphoreType.DMA((NTILES, 2)),
            pltpu.SemaphoreType.DMA((NTILES, 2)),
            pltpu.SemaphoreType.DMA((NTILES, 2)),
            pltpu.SemaphoreType.DMA((NTILES, 2)),
        ],
    )
    return pl.pallas_call(
        body,
        grid_spec=grid_spec,
        out_shape=jax.ShapeDtypeStruct((M, N), jnp.float32),
        compiler_params=pltpu.CompilerParams(
            collective_id=0,
            vmem_limit_bytes=60 * 1024 * 1024,
        ),
    )(my_y[None].astype(jnp.int32), dy, W)
